# baseline (speedup 1.0000x reference)
import functools

import jax
import jax.numpy as jnp
from jax import lax
from jax.experimental import pallas as pl
from jax.experimental.pallas import tpu as pltpu
from jax.experimental.pallas import tpu_sc as plsc

N = 8
DEPTH = 3
DIM = 1024
BLK = 1024

PREC = jax.lax.Precision.DEFAULT

_NW = 32
_K = 48

def _nt(a, b):
    return jax.lax.dot_general(a, b, (((1,), (1,)), ((), ())),
                               preferred_element_type=jnp.float32,
                               precision=PREC)


def _nn(a, b):
    return jax.lax.dot_general(a, b, (((1,), (0,)), ((), ())),
                               preferred_element_type=jnp.float32,
                               precision=PREC)


def _route_kernel(x_ref, w0_ref, w1_ref, w2_ref, ws_ref, idx_ref):
    f32 = jnp.float32
    xb = x_ref[...]
    T = xb.shape[0]
    NEG = f32(-1e30)
    cols8 = jax.lax.broadcasted_iota(jnp.int32, (T, N), 1)

    a0 = _nt(xb, w0_ref[...])
    m0 = jnp.max(a0, axis=1, keepdims=True)
    k0 = jnp.min(jnp.where(a0 == m0, cols8, N), axis=1, keepdims=True)
    oh0 = (cols8 == k0).astype(f32)
    x1 = xb + _nn(oh0, ws_ref[...])

    a1 = _nt(x1, w1_ref[...])
    cols64 = jax.lax.broadcasted_iota(jnp.int32, (T, N * N), 1)
    a1m = jnp.where((cols64 >> 3) == k0, a1, NEG)
    m1 = jnp.max(a1m, axis=1, keepdims=True)
    c1 = jnp.min(jnp.where(a1m == m1, cols64, N * N), axis=1, keepdims=True)
    x2 = x1 + _nn((cols8 == (c1 & 7)).astype(f32), ws_ref[...])

    a2 = _nt(x2, w2_ref[...])
    cols512 = jax.lax.broadcasted_iota(jnp.int32, (T, N * N * N), 1)
    a2m = jnp.where((cols512 >> 3) == c1, a2, NEG)
    m2 = jnp.max(a2m, axis=1, keepdims=True)
    c2 = jnp.min(jnp.where(a2m == m2, cols512, N * N * N), axis=1,
                 keepdims=True)

    idx_ref[0] = jnp.concatenate([k0, c1 + N, c2 + (N + N * N)], axis=1)


def _route(x, w0, w1, w2, ws):
    B = x.shape[0]
    grid = (B // BLK,)
    full = lambda shape: pl.BlockSpec(shape, lambda i: (0,) * len(shape))
    return pl.pallas_call(
        _route_kernel,
        grid=grid,
        in_specs=[
            pl.BlockSpec((BLK, DIM), lambda i: (i, 0)),
            full((N, DIM)), full((N * N, DIM)), full((N * N * N, DIM)),
            full((N, DIM)),
        ],
        out_specs=pl.BlockSpec((1, BLK, DEPTH), lambda i: (i, 0, 0)),
        out_shape=jax.ShapeDtypeStruct((B // BLK, BLK, DEPTH), jnp.int32),
    )(x, w0, w1, w2, ws)


def _make_sc_gather(rows_total, nch):
    mesh = plsc.VectorSubcoreMesh(core_axis_name="c", subcore_axis_name="s")

    @functools.partial(
        pl.kernel, mesh=mesh,
        out_type=jax.ShapeDtypeStruct((rows_total, DIM), jnp.float32),
        scratch_types=[
            pltpu.VMEM((nch, _K), jnp.int32),
            pltpu.VMEM((_K, DIM), jnp.float32),
            pltpu.VMEM((_K, DIM), jnp.float32),
            pltpu.SemaphoreType.DMA,
            pltpu.SemaphoreType.DMA,
        ],
    )
    def sc_gather(vtab_hbm, idx_hbm, out_hbm, idx_v, rows0, rows1,
                  sem0, sem1):
        wid = lax.axis_index("s") * 2 + lax.axis_index("c")
        pltpu.sync_copy(idx_hbm.at[wid], idx_v)
        base = wid * (nch * _K)
        bufs = (rows0, rows1)
        sems = (sem0, sem1)
        cps = [None, None]
        cps[0] = pltpu.async_copy(vtab_hbm.at[idx_v.at[0]], rows0, sem0)
        for j in range(nch):
            if j + 1 < nch:
                cps[(j + 1) % 2] = pltpu.async_copy(
                    vtab_hbm.at[idx_v.at[j + 1]], bufs[(j + 1) % 2],
                    sems[(j + 1) % 2])
            cps[j % 2].wait()
            pltpu.sync_copy(bufs[j % 2], out_hbm.at[pl.ds(base + j * _K, _K)])

    return sc_gather


def kernel(x, level0_data, level0_values, level1_data, level1_values,
           level2_data, level2_values, W_state):
    B = x.shape[0]
    w0 = level0_data.reshape(N, DIM)
    w1 = level1_data.reshape(N * N, DIM)
    w2 = level2_data.reshape(N * N * N, DIM)
    vtab = jnp.concatenate([level0_values.reshape(N, DIM),
                            level1_values.reshape(N * N, DIM),
                            level2_values.reshape(N * N * N, DIM)],
                           axis=0).astype(jnp.bfloat16).astype(jnp.float32)

    idx = _route(x, w0, w1, w2, W_state)
    rows_total = DEPTH * B
    nch = rows_total // (_NW * _K)
    idx_flat = idx.reshape(B, DEPTH).T.reshape(_NW, nch, _K)
    out = _make_sc_gather(rows_total, nch)(vtab, idx_flat)
    return out.reshape(DEPTH, B, 1, DIM)


# speedup vs baseline: 3.7247x; 3.7247x over previous
import jax
import jax.numpy as jnp
from jax.experimental import pallas as pl

N = 8
DEPTH = 3
DIM = 1024
BLK = 1024

PREC = jax.lax.Precision.DEFAULT


def _nt(a, b):
    return jax.lax.dot_general(a, b, (((1,), (1,)), ((), ())),
                               preferred_element_type=jnp.float32,
                               precision=PREC)


def _nn(a, b):
    return jax.lax.dot_general(a, b, (((1,), (0,)), ((), ())),
                               preferred_element_type=jnp.float32,
                               precision=PREC)


def _tree_kernel(x_ref, w0_ref, w1_ref, w2_ref, v0_ref, v1_ref, v2_ref,
                 ws_ref, out_ref):
    f32 = jnp.float32
    xb = x_ref[...]
    T = xb.shape[0]
    NEG = f32(-1e30)
    cols8 = jax.lax.broadcasted_iota(jnp.int32, (T, N), 1)

    a0 = _nt(xb, w0_ref[...])
    m0 = jnp.max(a0, axis=1, keepdims=True)
    k0 = jnp.min(jnp.where(a0 == m0, cols8, N), axis=1, keepdims=True)
    oh0 = (cols8 == k0).astype(f32)
    out_ref[0, :, 0, :] = _nn(oh0, v0_ref[...])
    x1 = xb + _nn(oh0, ws_ref[...])

    a1 = _nt(x1, w1_ref[...])
    cols64 = jax.lax.broadcasted_iota(jnp.int32, (T, N * N), 1)
    a1m = jnp.where((cols64 >> 3) == k0, a1, NEG)
    m1 = jnp.max(a1m, axis=1, keepdims=True)
    c1 = jnp.min(jnp.where(a1m == m1, cols64, N * N), axis=1, keepdims=True)
    oh1 = (cols64 == c1).astype(f32)
    out_ref[1, :, 0, :] = _nn(oh1, v1_ref[...])
    x2 = x1 + _nn((cols8 == (c1 & 7)).astype(f32), ws_ref[...])

    a2 = _nt(x2, w2_ref[...])
    cols512 = jax.lax.broadcasted_iota(jnp.int32, (T, N * N * N), 1)
    a2m = jnp.where((cols512 >> 3) == c1, a2, NEG)
    m2 = jnp.max(a2m, axis=1, keepdims=True)
    c2 = jnp.min(jnp.where(a2m == m2, cols512, N * N * N), axis=1,
                 keepdims=True)
    oh2 = (cols512 == c2).astype(f32)
    out_ref[2, :, 0, :] = _nn(oh2, v2_ref[...])


def kernel(x, level0_data, level0_values, level1_data, level1_values,
           level2_data, level2_values, W_state):
    B = x.shape[0]
    w0 = level0_data.reshape(N, DIM)
    w1 = level1_data.reshape(N * N, DIM)
    w2 = level2_data.reshape(N * N * N, DIM)
    v0 = level0_values.reshape(N, DIM)
    v1 = level1_values.reshape(N * N, DIM)
    v2 = level2_values.reshape(N * N * N, DIM)

    grid = (B // BLK,)
    full = lambda shape: pl.BlockSpec(shape, lambda i: (0,) * len(shape))
    out = pl.pallas_call(
        _tree_kernel,
        grid=grid,
        in_specs=[
            pl.BlockSpec((BLK, DIM), lambda i: (i, 0)),
            full((N, DIM)), full((N * N, DIM)), full((N * N * N, DIM)),
            full((N, DIM)), full((N * N, DIM)), full((N * N * N, DIM)),
            full((N, DIM)),
        ],
        out_specs=pl.BlockSpec((DEPTH, BLK, 1, DIM), lambda i: (0, i, 0, 0)),
        out_shape=jax.ShapeDtypeStruct((DEPTH, B, 1, DIM), jnp.float32),
    )(x, w0, w1, w2, v0, v1, v2, W_state)
    return out
